# baseline (speedup 1.0000x reference)
import functools

import jax
import jax.numpy as jnp
from jax import lax
from jax.experimental import pallas as pl
from jax.experimental.pallas import tpu as pltpu
from jax.experimental.pallas import tpu_sc as plsc


def _score_body(hs_ref, w_in_ref, b_in_ref, embed_ref, ind_ref):
    f32 = jnp.float32
    hs = hs_ref[0]
    x = jnp.dot(w_in_ref[...], hs, preferred_element_type=f32)
    x = x + b_in_ref[...]
    s = jnp.dot(embed_ref[...], x, preferred_element_type=f32)
    e2 = jnp.sum(embed_ref[...] * embed_ref[...], axis=1, keepdims=True)
    score = 2.0 * s - e2
    k = score.shape[0]
    mx = jnp.max(score, axis=0, keepdims=True)
    idx = jax.lax.broadcasted_iota(jnp.int32, score.shape, 0)
    ind = jnp.min(jnp.where(score == mx, idx, k), axis=0, keepdims=True)
    ind_ref[0] = ind


def _out_body(q_ref, w_out_ref, b_out_ref, out_ref):
    d = w_out_ref.shape[1]
    q = q_ref[0][:, :d]
    out = lax.dot_general(w_out_ref[...], q,
                          (((1,), (1,)), ((), ())),
                          preferred_element_type=jnp.float32)
    out_ref[0] = out + b_out_ref[...]


_SC_INFO = plsc.get_sparse_core_info()
_NC = _SC_INFO.num_cores
_NW = _NC * _SC_INFO.num_subcores


def _sc_gather(table, idx):
    bt = idx.shape[0]
    d = table.shape[1]
    b_per_w = bt // _NW
    n_chunks = 4
    chunk = b_per_w // n_chunks
    mesh = plsc.VectorSubcoreMesh(core_axis_name="c", subcore_axis_name="s")

    @functools.partial(
        pl.kernel, mesh=mesh,
        out_type=jax.ShapeDtypeStruct((bt, d), jnp.float32),
        scratch_types=[
            pltpu.VMEM((chunk,), jnp.int32),
            pltpu.VMEM((chunk, d), jnp.float32),
            pltpu.SemaphoreType.DMA,
        ],
    )
    def gather_kernel(table_hbm, idx_hbm, out_hbm, idx_v, rows_v, sem):
        wid = lax.axis_index("s") * _NC + lax.axis_index("c")
        base = wid * b_per_w
        for c in range(n_chunks):
            off = base + c * chunk
            pltpu.sync_copy(idx_hbm.at[pl.ds(off, chunk)], idx_v)
            pltpu.async_copy(table_hbm.at[idx_v], rows_v, sem).wait()
            pltpu.sync_copy(rows_v, out_hbm.at[pl.ds(off, chunk)])

    return gather_kernel(table, idx)


@functools.partial(jax.jit, static_argnames=())
def kernel(hidden_states, W_in, b_in, embed, W_out, b_out):
    B, H, T = hidden_states.shape
    D = W_in.shape[0]
    K = embed.shape[0]

    b_in_c = b_in.reshape(D, 1)
    b_out_c = b_out.reshape(H, 1)
    rep = lambda *_: (0, 0)

    ind = pl.pallas_call(
        _score_body,
        grid=(B,),
        in_specs=[
            pl.BlockSpec((1, H, T), lambda b: (b, 0, 0)),
            pl.BlockSpec((D, H), rep),
            pl.BlockSpec((D, 1), rep),
            pl.BlockSpec((K, D), rep),
        ],
        out_specs=pl.BlockSpec((1, 1, T), lambda b: (b, 0, 0)),
        out_shape=jax.ShapeDtypeStruct((B, 1, T), jnp.int32),
        compiler_params=pltpu.CompilerParams(
            dimension_semantics=("parallel",)),
    )(hidden_states, W_in, b_in_c, embed)

    DP = 128
    embed_pad = jnp.pad(embed, ((0, 0), (0, DP - D)))
    quant = _sc_gather(embed_pad, ind.reshape(B * T))

    out = pl.pallas_call(
        _out_body,
        grid=(B,),
        in_specs=[
            pl.BlockSpec((1, T, DP), lambda b: (b, 0, 0)),
            pl.BlockSpec((H, D), rep),
            pl.BlockSpec((H, 1), rep),
        ],
        out_specs=pl.BlockSpec((1, H, T), lambda b: (b, 0, 0)),
        out_shape=jax.ShapeDtypeStruct((B, H, T), jnp.float32),
        compiler_params=pltpu.CompilerParams(
            dimension_semantics=("parallel",)),
    )(quant.reshape(B, T, DP), W_out, b_out_c)
    return out


# speedup vs baseline: 1.6926x; 1.6926x over previous
import functools

import jax
import jax.numpy as jnp
from jax import lax
from jax.experimental import pallas as pl
from jax.experimental.pallas import tpu as pltpu
from jax.experimental.pallas import tpu_sc as plsc


def _score_body(hs_ref, w_in_ref, b_in_ref, embed_ref, ind_ref):
    f32 = jnp.float32
    hs = hs_ref[0]
    x = jnp.dot(w_in_ref[...], hs, preferred_element_type=f32)
    x = x + b_in_ref[...]
    s = jnp.dot(embed_ref[...], x, preferred_element_type=f32)
    e2 = jnp.sum(embed_ref[...] * embed_ref[...], axis=1, keepdims=True)
    score = 2.0 * s - e2
    k = score.shape[0]
    mx = jnp.max(score, axis=0, keepdims=True)
    idx = jax.lax.broadcasted_iota(jnp.int32, score.shape, 0)
    ind = jnp.min(jnp.where(score == mx, idx, k), axis=0, keepdims=True)
    ind_ref[0] = ind


def _out_body(q_ref, w_out_ref, b_out_ref, out_ref):
    d = w_out_ref.shape[1]
    q = q_ref[0][:, :d]
    out = lax.dot_general(w_out_ref[...], q,
                          (((1,), (1,)), ((), ())),
                          preferred_element_type=jnp.float32)
    out_ref[0] = out + b_out_ref[...]


_SC_INFO = plsc.get_sparse_core_info()
_NC = _SC_INFO.num_cores
_NW = _NC * _SC_INFO.num_subcores


def _sc_gather(table, idx):
    bt = idx.shape[0]
    d = table.shape[1]
    b_per_w = bt // _NW
    n_chunks = 4
    chunk = b_per_w // n_chunks
    mesh = plsc.VectorSubcoreMesh(core_axis_name="c", subcore_axis_name="s")

    @functools.partial(
        pl.kernel, mesh=mesh,
        out_type=jax.ShapeDtypeStruct((bt, d), jnp.float32),
        scratch_types=[
            pltpu.VMEM((chunk,), jnp.int32),
            pltpu.VMEM((chunk, d), jnp.float32),
            pltpu.VMEM_SHARED((table.shape[0], d), jnp.float32),
            pltpu.SemaphoreType.DMA,
        ],
    )
    def gather_kernel(table_hbm, idx_hbm, out_hbm, idx_v, rows_v, table_sp,
                      sem):
        sid = lax.axis_index("s")
        wid = sid * _NC + lax.axis_index("c")
        base = wid * b_per_w
        @pl.when(sid == 0)
        def _stage():
            pltpu.sync_copy(table_hbm, table_sp)
        plsc.subcore_barrier()
        for c in range(n_chunks):
            off = base + c * chunk
            pltpu.sync_copy(idx_hbm.at[pl.ds(off, chunk)], idx_v)
            pltpu.async_copy(table_sp.at[idx_v], rows_v, sem).wait()
            pltpu.sync_copy(rows_v, out_hbm.at[pl.ds(off, chunk)])

    return gather_kernel(table, idx)


@functools.partial(jax.jit, static_argnames=())
def kernel(hidden_states, W_in, b_in, embed, W_out, b_out):
    B, H, T = hidden_states.shape
    D = W_in.shape[0]
    K = embed.shape[0]

    b_in_c = b_in.reshape(D, 1)
    b_out_c = b_out.reshape(H, 1)
    rep = lambda *_: (0, 0)

    ind = pl.pallas_call(
        _score_body,
        grid=(B,),
        in_specs=[
            pl.BlockSpec((1, H, T), lambda b: (b, 0, 0)),
            pl.BlockSpec((D, H), rep),
            pl.BlockSpec((D, 1), rep),
            pl.BlockSpec((K, D), rep),
        ],
        out_specs=pl.BlockSpec((1, 1, T), lambda b: (b, 0, 0)),
        out_shape=jax.ShapeDtypeStruct((B, 1, T), jnp.int32),
        compiler_params=pltpu.CompilerParams(
            dimension_semantics=("parallel",)),
    )(hidden_states, W_in, b_in_c, embed)

    DP = 128
    embed_pad = jnp.pad(embed, ((0, 0), (0, DP - D)))
    quant = _sc_gather(embed_pad, ind.reshape(B * T))

    out = pl.pallas_call(
        _out_body,
        grid=(B,),
        in_specs=[
            pl.BlockSpec((1, T, DP), lambda b: (b, 0, 0)),
            pl.BlockSpec((H, D), rep),
            pl.BlockSpec((H, 1), rep),
        ],
        out_specs=pl.BlockSpec((1, H, T), lambda b: (b, 0, 0)),
        out_shape=jax.ShapeDtypeStruct((B, H, T), jnp.float32),
        compiler_params=pltpu.CompilerParams(
            dimension_semantics=("parallel",)),
    )(quant.reshape(B, T, DP), W_out, b_out_c)
    return out
